# Initial kernel scaffold; baseline (speedup 1.0000x reference)
import jax
import jax.numpy as jnp
from jax.experimental import pallas as pl


def kernel(x, edge_index, batch, W1_l, W1_r, W2_l, W2_r, Wfc):
    raise NotImplementedError("write your pallas kernel here")


# speedup vs baseline: 12.5024x; 12.5024x over previous
import functools

import jax
import jax.numpy as jnp
from jax import lax
from jax.experimental import pallas as pl
from jax.experimental.pallas import tpu as pltpu
from jax.experimental.pallas import tpu_sc as plsc

N = 10000
E = 320000
F_IN = 128
DIM = 16
OUT = 10
G = 64

NC = 2
NS = 16
NW = NC * NS
CHUNK = 128
NCH = 80
EPW = NCH * CHUNK
EP = NW * EPW
NPAD = 10240
RPS = NPAD // NS


def _proj1_body(x_ref, w_ref, yl_ref, yr_ref):
    y = lax.dot_general(x_ref[...], w_ref[...], (((1,), (1,)), ((), ())),
                        preferred_element_type=jnp.float32)
    yl_ref[...] = y[:, :DIM]
    yr_ref[...] = y[:, DIM:]


def _combine2_body(acc_ref, cntp_ref, yr_ref, w_ref, zl_ref, zr_ref, invc_ref):
    acc = acc_ref[0] + acc_ref[1]
    cnt = cntp_ref[0] + cntp_ref[1]
    invc = (1.0 / jnp.maximum(cnt, 1.0))[:N, None]
    h = jax.nn.relu(acc[:N] * invc + yr_ref[...])
    z = lax.dot_general(h, w_ref[...], (((1,), (1,)), ((), ())),
                        preferred_element_type=jnp.float32)
    zl_ref[...] = z[:, :DIM]
    zr_ref[...] = z[:, DIM:]
    invc_ref[...] = invc


def _final_body(acc_ref, invc_ref, zr_ref, b_ref, wfc_ref, out_ref):
    acc = acc_ref[0] + acc_ref[1]
    h2 = acc[:N] * invc_ref[...] + zr_ref[...]
    onehot = (lax.broadcasted_iota(jnp.int32, (G, N), 0) == b_ref[...]
              ).astype(jnp.float32)
    sums = lax.dot_general(onehot, h2, (((1,), (0,)), ((), ())),
                           preferred_element_type=jnp.float32)
    cnt_g = jnp.sum(onehot, axis=1)
    pooled = sums * (1.0 / jnp.maximum(cnt_g, 1.0))[:, None]
    logits = lax.dot_general(pooled, wfc_ref[...], (((1,), (1,)), ((), ())),
                             preferred_element_type=jnp.float32)
    out_ref[...] = jax.nn.sigmoid(logits)


_SC_MESH = plsc.VectorSubcoreMesh(core_axis_name="c", subcore_axis_name="s")
_SC_PARAMS = pltpu.CompilerParams(use_tc_tiling_on_sc=False)


def _edge_loop(y_hbm, idx_src, idx_dst, rows, acc, sem):
    def step(j, carry):
        pltpu.async_copy(y_hbm.at[idx_src.at[j]], rows, sem).wait()
        pltpu.sync_copy(rows, acc.at[idx_dst.at[j]], add=True)
        return carry
    lax.fori_loop(0, NCH, step, 0)


def _agg_a_body(y_hbm, src3, dst3, zeros_hbm, rows_out, cnt_out,
                idx_src, idx_dst, ones_buf, zbuf, rows, acc, cnt_acc, sem):
    c = lax.axis_index("c")
    s = lax.axis_index("s")
    wid = c * NS + s
    pltpu.sync_copy(src3.at[wid], idx_src)
    pltpu.sync_copy(dst3.at[wid], idx_dst)
    pltpu.sync_copy(zeros_hbm.at[pl.ds(s * RPS, RPS)],
                    acc.at[pl.ds(s * RPS, RPS)])
    z16 = jnp.zeros((16,), jnp.float32)
    ones16 = jnp.ones((16,), jnp.float32)

    def fill(i, carry):
        zbuf[pl.ds(i * 16, 16)] = z16
        return carry
    lax.fori_loop(0, RPS // 16, fill, 0)

    def fill1(i, carry):
        ones_buf[pl.ds(i * 16, 16)] = ones16
        return carry
    lax.fori_loop(0, CHUNK // 16, fill1, 0)
    pltpu.sync_copy(zbuf, cnt_acc.at[pl.ds(s * RPS, RPS)])
    plsc.subcore_barrier()

    def step(j, carry):
        pltpu.async_copy(y_hbm.at[idx_src.at[j]], rows, sem).wait()
        pltpu.sync_copy(rows, acc.at[idx_dst.at[j]], add=True)
        pltpu.sync_copy(ones_buf, cnt_acc.at[idx_dst.at[j]], add=True)
        return carry
    lax.fori_loop(0, NCH, step, 0)
    plsc.subcore_barrier()

    r0 = s * RPS
    pltpu.sync_copy(acc.at[pl.ds(r0, RPS)],
                    rows_out.at[pl.ds(c * NPAD + r0, RPS)])
    pltpu.sync_copy(cnt_acc.at[pl.ds(r0, RPS)],
                    cnt_out.at[pl.ds(c * NPAD + r0, RPS)])


def _agg_b_body(y_hbm, src3, dst3, zeros_hbm, rows_out,
                idx_src, idx_dst, rows, acc, sem):
    c = lax.axis_index("c")
    s = lax.axis_index("s")
    wid = c * NS + s
    pltpu.sync_copy(src3.at[wid], idx_src)
    pltpu.sync_copy(dst3.at[wid], idx_dst)
    pltpu.sync_copy(zeros_hbm.at[pl.ds(s * RPS, RPS)],
                    acc.at[pl.ds(s * RPS, RPS)])
    plsc.subcore_barrier()

    _edge_loop(y_hbm, idx_src, idx_dst, rows, acc, sem)
    plsc.subcore_barrier()

    r0 = s * RPS
    pltpu.sync_copy(acc.at[pl.ds(r0, RPS)],
                    rows_out.at[pl.ds(c * NPAD + r0, RPS)])


_agg_a = pl.kernel(
    _agg_a_body,
    out_type=[
        jax.ShapeDtypeStruct((NC * NPAD, DIM), jnp.float32),
        jax.ShapeDtypeStruct((NC * NPAD,), jnp.float32),
    ],
    mesh=_SC_MESH,
    scratch_types=[
        pltpu.VMEM((NCH, CHUNK), jnp.int32),
        pltpu.VMEM((NCH, CHUNK), jnp.int32),
        pltpu.VMEM((CHUNK,), jnp.float32),
        pltpu.VMEM((RPS,), jnp.float32),
        pltpu.VMEM((CHUNK, DIM), jnp.float32),
        pltpu.VMEM_SHARED((NPAD, DIM), jnp.float32),
        pltpu.VMEM_SHARED((NPAD,), jnp.float32),
        pltpu.SemaphoreType.DMA,
    ],
    compiler_params=_SC_PARAMS,
)

_agg_b = pl.kernel(
    _agg_b_body,
    out_type=jax.ShapeDtypeStruct((NC * NPAD, DIM), jnp.float32),
    mesh=_SC_MESH,
    scratch_types=[
        pltpu.VMEM((NCH, CHUNK), jnp.int32),
        pltpu.VMEM((NCH, CHUNK), jnp.int32),
        pltpu.VMEM((CHUNK, DIM), jnp.float32),
        pltpu.VMEM_SHARED((NPAD, DIM), jnp.float32),
        pltpu.SemaphoreType.DMA,
    ],
    compiler_params=_SC_PARAMS,
)

_proj1 = pl.pallas_call(
    _proj1_body,
    out_shape=[
        jax.ShapeDtypeStruct((N, DIM), jnp.float32),
        jax.ShapeDtypeStruct((N, DIM), jnp.float32),
    ],
)

_combine2 = pl.pallas_call(
    _combine2_body,
    out_shape=[
        jax.ShapeDtypeStruct((N, DIM), jnp.float32),
        jax.ShapeDtypeStruct((N, DIM), jnp.float32),
        jax.ShapeDtypeStruct((N, 1), jnp.float32),
    ],
)

_final = pl.pallas_call(
    _final_body,
    out_shape=jax.ShapeDtypeStruct((G, OUT), jnp.float32),
)


@jax.jit
def kernel(x, edge_index, batch, W1_l, W1_r, W2_l, W2_r, Wfc):
    src = edge_index[0]
    dst = edge_index[1]
    pad = EP - E
    src_p = jnp.concatenate([src, jnp.zeros((pad,), jnp.int32)])
    dst_p = jnp.concatenate([dst, jnp.full((pad,), N, jnp.int32)])
    src3 = src_p.reshape(NW, NCH, CHUNK)
    dst3 = dst_p.reshape(NW, NCH, CHUNK)
    zeros_pad = jnp.zeros((NPAD, DIM), jnp.float32)
    w1 = jnp.concatenate([W1_l, W1_r], axis=0)
    w2 = jnp.concatenate([W2_l, W2_r], axis=0)

    y_l, y_r = _proj1(x, w1)
    rows1, cntp = _agg_a(y_l, src3, dst3, zeros_pad)
    rows1 = rows1.reshape(NC, NPAD, DIM)
    cntp = cntp.reshape(NC, NPAD)
    z_l, z_r, invc = _combine2(rows1, cntp, y_r, w2)
    rows2 = _agg_b(z_l, src3, dst3, zeros_pad).reshape(NC, NPAD, DIM)
    return _final(rows2, invc, z_r, batch.reshape(1, N), Wfc)


# speedup vs baseline: 17.3302x; 1.3861x over previous
import functools

import jax
import jax.numpy as jnp
from jax import lax
from jax.experimental import pallas as pl
from jax.experimental.pallas import tpu as pltpu
from jax.experimental.pallas import tpu_sc as plsc

N = 10000
E = 320000
F_IN = 128
DIM = 16
OUT = 10
G = 64

NC = 2
NS = 16
NW = NC * NS
CHUNK = 128
NCH = 80
EPW = NCH * CHUNK
EP = NW * EPW
NPAD = 10240
RPS = NPAD // NS


def _proj1_body(x_ref, w_ref, yl_ref, yr_ref):
    y = lax.dot_general(x_ref[...], w_ref[...], (((1,), (1,)), ((), ())),
                        preferred_element_type=jnp.float32)
    yl_ref[...] = y[:, :DIM]
    yr_ref[...] = y[:, DIM:]


def _combine2_body(acc_ref, cntp_ref, yr_ref, w_ref, zl_ref, zr_ref, invc_ref):
    acc = acc_ref[0] + acc_ref[1]
    cnt = cntp_ref[0] + cntp_ref[1]
    invc = (1.0 / jnp.maximum(cnt, 1.0))[:N, None]
    h = jax.nn.relu(acc[:N] * invc + yr_ref[...])
    z = lax.dot_general(h, w_ref[...], (((1,), (1,)), ((), ())),
                        preferred_element_type=jnp.float32)
    zl_ref[...] = z[:, :DIM]
    zr_ref[...] = z[:, DIM:]
    invc_ref[...] = invc


def _final_body(acc_ref, invc_ref, zr_ref, b_ref, wfc_ref, out_ref):
    acc = acc_ref[0] + acc_ref[1]
    h2 = acc[:N] * invc_ref[...] + zr_ref[...]
    onehot = (lax.broadcasted_iota(jnp.int32, (G, N), 0) == b_ref[...]
              ).astype(jnp.float32)
    sums = lax.dot_general(onehot, h2, (((1,), (0,)), ((), ())),
                           preferred_element_type=jnp.float32)
    cnt_g = jnp.sum(onehot, axis=1)
    pooled = sums * (1.0 / jnp.maximum(cnt_g, 1.0))[:, None]
    logits = lax.dot_general(pooled, wfc_ref[...], (((1,), (1,)), ((), ())),
                             preferred_element_type=jnp.float32)
    out_ref[...] = jax.nn.sigmoid(logits)


_SC_MESH = plsc.VectorSubcoreMesh(core_axis_name="c", subcore_axis_name="s")
_SC_PARAMS = pltpu.CompilerParams(use_tc_tiling_on_sc=False)


NBUF = 4


def _edge_loop(y_hbm, idx_src, idx_dst, rows, acc, sems,
               ones_buf=None, cnt_acc=None):
    for b in range(NBUF):
        pltpu.async_copy(y_hbm.at[idx_src.at[b]], rows.at[b], sems[b])

    dummy = y_hbm.at[pl.ds(0, CHUNK)]

    def step(g, carry):
        for b in range(NBUF):
            j = g * NBUF + b
            pltpu.make_async_copy(dummy, rows.at[b], sems[b]).wait()
            pltpu.sync_copy(rows.at[b], acc.at[idx_dst.at[j]], add=True)
            if cnt_acc is not None:
                pltpu.sync_copy(ones_buf, cnt_acc.at[idx_dst.at[j]], add=True)
            jn = jnp.minimum(j + NBUF, NCH - 1)
            pltpu.async_copy(y_hbm.at[idx_src.at[jn]], rows.at[b], sems[b])
        return carry

    lax.fori_loop(0, NCH // NBUF, step, 0)
    for b in range(NBUF):
        pltpu.make_async_copy(dummy, rows.at[b], sems[b]).wait()


def _agg_a_body(y_hbm, src3, dst3, zeros_hbm, rows_out, cnt_out,
                idx_src, idx_dst, ones_buf, zbuf, rows, acc, cnt_acc, *sems):
    c = lax.axis_index("c")
    s = lax.axis_index("s")
    wid = c * NS + s
    pltpu.sync_copy(src3.at[wid], idx_src)
    pltpu.sync_copy(dst3.at[wid], idx_dst)
    pltpu.sync_copy(zeros_hbm.at[pl.ds(s * RPS, RPS)],
                    acc.at[pl.ds(s * RPS, RPS)])
    z16 = jnp.zeros((16,), jnp.float32)
    ones16 = jnp.ones((16,), jnp.float32)

    def fill(i, carry):
        zbuf[pl.ds(i * 16, 16)] = z16
        return carry
    lax.fori_loop(0, RPS // 16, fill, 0)

    def fill1(i, carry):
        ones_buf[pl.ds(i * 16, 16)] = ones16
        return carry
    lax.fori_loop(0, CHUNK // 16, fill1, 0)
    pltpu.sync_copy(zbuf, cnt_acc.at[pl.ds(s * RPS, RPS)])
    plsc.subcore_barrier()

    _edge_loop(y_hbm, idx_src, idx_dst, rows, acc, sems,
               ones_buf=ones_buf, cnt_acc=cnt_acc)
    plsc.subcore_barrier()

    r0 = s * RPS
    pltpu.sync_copy(acc.at[pl.ds(r0, RPS)],
                    rows_out.at[pl.ds(c * NPAD + r0, RPS)])
    pltpu.sync_copy(cnt_acc.at[pl.ds(r0, RPS)],
                    cnt_out.at[pl.ds(c * NPAD + r0, RPS)])


def _agg_b_body(y_hbm, src3, dst3, zeros_hbm, rows_out,
                idx_src, idx_dst, rows, acc, *sems):
    c = lax.axis_index("c")
    s = lax.axis_index("s")
    wid = c * NS + s
    pltpu.sync_copy(src3.at[wid], idx_src)
    pltpu.sync_copy(dst3.at[wid], idx_dst)
    pltpu.sync_copy(zeros_hbm.at[pl.ds(s * RPS, RPS)],
                    acc.at[pl.ds(s * RPS, RPS)])
    plsc.subcore_barrier()

    _edge_loop(y_hbm, idx_src, idx_dst, rows, acc, sems)
    plsc.subcore_barrier()

    r0 = s * RPS
    pltpu.sync_copy(acc.at[pl.ds(r0, RPS)],
                    rows_out.at[pl.ds(c * NPAD + r0, RPS)])


_agg_a = pl.kernel(
    _agg_a_body,
    out_type=[
        jax.ShapeDtypeStruct((NC * NPAD, DIM), jnp.float32),
        jax.ShapeDtypeStruct((NC * NPAD,), jnp.float32),
    ],
    mesh=_SC_MESH,
    scratch_types=[
        pltpu.VMEM((NCH, CHUNK), jnp.int32),
        pltpu.VMEM((NCH, CHUNK), jnp.int32),
        pltpu.VMEM((CHUNK,), jnp.float32),
        pltpu.VMEM((RPS,), jnp.float32),
        pltpu.VMEM((NBUF, CHUNK, DIM), jnp.float32),
        pltpu.VMEM_SHARED((NPAD, DIM), jnp.float32),
        pltpu.VMEM_SHARED((NPAD,), jnp.float32),
    ] + [pltpu.SemaphoreType.DMA] * NBUF,
    compiler_params=_SC_PARAMS,
)

_agg_b = pl.kernel(
    _agg_b_body,
    out_type=jax.ShapeDtypeStruct((NC * NPAD, DIM), jnp.float32),
    mesh=_SC_MESH,
    scratch_types=[
        pltpu.VMEM((NCH, CHUNK), jnp.int32),
        pltpu.VMEM((NCH, CHUNK), jnp.int32),
        pltpu.VMEM((NBUF, CHUNK, DIM), jnp.float32),
        pltpu.VMEM_SHARED((NPAD, DIM), jnp.float32),
    ] + [pltpu.SemaphoreType.DMA] * NBUF,
    compiler_params=_SC_PARAMS,
)

_proj1 = pl.pallas_call(
    _proj1_body,
    out_shape=[
        jax.ShapeDtypeStruct((N, DIM), jnp.float32),
        jax.ShapeDtypeStruct((N, DIM), jnp.float32),
    ],
)

_combine2 = pl.pallas_call(
    _combine2_body,
    out_shape=[
        jax.ShapeDtypeStruct((N, DIM), jnp.float32),
        jax.ShapeDtypeStruct((N, DIM), jnp.float32),
        jax.ShapeDtypeStruct((N, 1), jnp.float32),
    ],
)

_final = pl.pallas_call(
    _final_body,
    out_shape=jax.ShapeDtypeStruct((G, OUT), jnp.float32),
)


@jax.jit
def kernel(x, edge_index, batch, W1_l, W1_r, W2_l, W2_r, Wfc):
    src = edge_index[0]
    dst = edge_index[1]
    pad = EP - E
    src_p = jnp.concatenate([src, jnp.zeros((pad,), jnp.int32)])
    dst_p = jnp.concatenate([dst, jnp.full((pad,), N, jnp.int32)])
    src3 = src_p.reshape(NW, NCH, CHUNK)
    dst3 = dst_p.reshape(NW, NCH, CHUNK)
    zeros_pad = jnp.zeros((NPAD, DIM), jnp.float32)
    w1 = jnp.concatenate([W1_l, W1_r], axis=0)
    w2 = jnp.concatenate([W2_l, W2_r], axis=0)

    y_l, y_r = _proj1(x, w1)
    rows1, cntp = _agg_a(y_l, src3, dst3, zeros_pad)
    rows1 = rows1.reshape(NC, NPAD, DIM)
    cntp = cntp.reshape(NC, NPAD)
    z_l, z_r, invc = _combine2(rows1, cntp, y_r, w2)
    rows2 = _agg_b(z_l, src3, dst3, zeros_pad).reshape(NC, NPAD, DIM)
    return _final(rows2, invc, z_r, batch.reshape(1, N), Wfc)


# speedup vs baseline: 24.9161x; 1.4377x over previous
import functools

import jax
import jax.numpy as jnp
from jax import lax
from jax.experimental import pallas as pl
from jax.experimental.pallas import tpu as pltpu
from jax.experimental.pallas import tpu_sc as plsc

N = 10000
E = 320000
F_IN = 128
DIM = 16
OUT = 10
G = 64

NC = 2
NS = 16
NW = NC * NS
CHUNK = 128
NCH = 80
EPW = NCH * CHUNK
EP = NW * EPW
NPAD = 10240
RPS = NPAD // NS


def _proj1_body(x_ref, w_ref, yl_ref, yr_ref):
    y = lax.dot_general(x_ref[...], w_ref[...], (((1,), (1,)), ((), ())),
                        preferred_element_type=jnp.float32)
    yl_ref[...] = y[:, :DIM]
    yr_ref[...] = y[:, DIM:]


def _combine2_body(acc_ref, cntp_ref, yr_ref, w_ref, zl_ref, zr_ref, invc_ref):
    acc = acc_ref[0] + acc_ref[1]
    cnt = cntp_ref[0] + cntp_ref[1]
    invc = (1.0 / jnp.maximum(cnt, 1.0))[:N, None]
    h = jax.nn.relu(acc[:N] * invc + yr_ref[...])
    z = lax.dot_general(h, w_ref[...], (((1,), (1,)), ((), ())),
                        preferred_element_type=jnp.float32)
    zl_ref[...] = z[:, :DIM]
    zr_ref[...] = z[:, DIM:]
    invc_ref[...] = invc


def _final_body(acc_ref, invc_ref, zr_ref, b_ref, wfc_ref, out_ref):
    acc = acc_ref[0] + acc_ref[1]
    h2 = acc[:N] * invc_ref[...] + zr_ref[...]
    onehot = (lax.broadcasted_iota(jnp.int32, (G, N), 0) == b_ref[...]
              ).astype(jnp.float32)
    sums = lax.dot_general(onehot, h2, (((1,), (0,)), ((), ())),
                           preferred_element_type=jnp.float32)
    cnt_g = jnp.sum(onehot, axis=1)
    pooled = sums * (1.0 / jnp.maximum(cnt_g, 1.0))[:, None]
    logits = lax.dot_general(pooled, wfc_ref[...], (((1,), (1,)), ((), ())),
                             preferred_element_type=jnp.float32)
    out_ref[...] = jax.nn.sigmoid(logits)


_SC_MESH = plsc.VectorSubcoreMesh(core_axis_name="c", subcore_axis_name="s")
_SC_PARAMS = pltpu.CompilerParams(use_tc_tiling_on_sc=False)


NBUF = 4


def _edge_loop(y_tbl, dummy_hbm, idx_src, idx_dst, rows, acc,
               gsems, ssems, ones_buf=None, cnt_acc=None, csems=None):
    for b in range(NBUF):
        pltpu.async_copy(y_tbl.at[idx_src.at[b]], rows.at[b], gsems[b])

    gd = dummy_hbm.at[pl.ds(0, CHUNK)]

    def step(g, carry):
        for b in range(NBUF):
            j = g * NBUF + b
            pltpu.make_async_copy(gd, rows.at[b], gsems[b]).wait()
            pltpu.async_copy(rows.at[b], acc.at[idx_dst.at[j]], ssems[b],
                             add=True)
            if cnt_acc is not None:
                pltpu.async_copy(ones_buf, cnt_acc.at[idx_dst.at[j]],
                                 csems[b], add=True)
        for b in range(NBUF):
            j = g * NBUF + b
            pltpu.make_async_copy(rows.at[b], acc.at[idx_dst.at[j]],
                                  ssems[b]).wait()
            if cnt_acc is not None:
                pltpu.make_async_copy(ones_buf, cnt_acc.at[idx_dst.at[j]],
                                      csems[b]).wait()
            jn = jnp.minimum(j + NBUF, NCH - 1)
            pltpu.async_copy(y_tbl.at[idx_src.at[jn]], rows.at[b], gsems[b])
        return carry

    lax.fori_loop(0, NCH // NBUF, step, 0)
    for b in range(NBUF):
        pltpu.make_async_copy(gd, rows.at[b], gsems[b]).wait()


def _agg_a_body(y_hbm, src3, dst3, zeros_hbm, rows_out, cnt_out,
                idx_src, idx_dst, ones_buf, zbuf, rows, y_sh, acc, cnt_acc,
                *sems):
    c = lax.axis_index("c")
    s = lax.axis_index("s")
    wid = c * NS + s
    pltpu.sync_copy(src3.at[wid], idx_src)
    pltpu.sync_copy(dst3.at[wid], idx_dst)
    pltpu.sync_copy(y_hbm.at[pl.ds(s * (N // NS), N // NS)],
                    y_sh.at[pl.ds(s * (N // NS), N // NS)])
    pltpu.sync_copy(zeros_hbm.at[pl.ds(s * RPS, RPS)],
                    acc.at[pl.ds(s * RPS, RPS)])
    z16 = jnp.zeros((16,), jnp.float32)
    ones16 = jnp.ones((16,), jnp.float32)

    def fill(i, carry):
        zbuf[pl.ds(i * 16, 16)] = z16
        return carry
    lax.fori_loop(0, RPS // 16, fill, 0)

    def fill1(i, carry):
        ones_buf[pl.ds(i * 16, 16)] = ones16
        return carry
    lax.fori_loop(0, CHUNK // 16, fill1, 0)
    pltpu.sync_copy(zbuf, cnt_acc.at[pl.ds(s * RPS, RPS)])
    plsc.subcore_barrier()

    _edge_loop(y_sh, y_hbm, idx_src, idx_dst, rows, acc,
               sems[:NBUF], sems[NBUF:2 * NBUF],
               ones_buf=ones_buf, cnt_acc=cnt_acc, csems=sems[2 * NBUF:])
    plsc.subcore_barrier()

    r0 = s * RPS
    pltpu.sync_copy(acc.at[pl.ds(r0, RPS)],
                    rows_out.at[pl.ds(c * NPAD + r0, RPS)])
    pltpu.sync_copy(cnt_acc.at[pl.ds(r0, RPS)],
                    cnt_out.at[pl.ds(c * NPAD + r0, RPS)])


def _agg_b_body(y_hbm, src3, dst3, zeros_hbm, rows_out,
                idx_src, idx_dst, rows, y_sh, acc, *sems):
    c = lax.axis_index("c")
    s = lax.axis_index("s")
    wid = c * NS + s
    pltpu.sync_copy(src3.at[wid], idx_src)
    pltpu.sync_copy(dst3.at[wid], idx_dst)
    pltpu.sync_copy(y_hbm.at[pl.ds(s * (N // NS), N // NS)],
                    y_sh.at[pl.ds(s * (N // NS), N // NS)])
    pltpu.sync_copy(zeros_hbm.at[pl.ds(s * RPS, RPS)],
                    acc.at[pl.ds(s * RPS, RPS)])
    plsc.subcore_barrier()

    _edge_loop(y_sh, y_hbm, idx_src, idx_dst, rows, acc,
               sems[:NBUF], sems[NBUF:])
    plsc.subcore_barrier()

    r0 = s * RPS
    pltpu.sync_copy(acc.at[pl.ds(r0, RPS)],
                    rows_out.at[pl.ds(c * NPAD + r0, RPS)])


_agg_a = pl.kernel(
    _agg_a_body,
    out_type=[
        jax.ShapeDtypeStruct((NC * NPAD, DIM), jnp.float32),
        jax.ShapeDtypeStruct((NC * NPAD,), jnp.float32),
    ],
    mesh=_SC_MESH,
    scratch_types=[
        pltpu.VMEM((NCH, CHUNK), jnp.int32),
        pltpu.VMEM((NCH, CHUNK), jnp.int32),
        pltpu.VMEM((CHUNK,), jnp.float32),
        pltpu.VMEM((RPS,), jnp.float32),
        pltpu.VMEM((NBUF, CHUNK, DIM), jnp.float32),
        pltpu.VMEM_SHARED((N, DIM), jnp.float32),
        pltpu.VMEM_SHARED((NPAD, DIM), jnp.float32),
        pltpu.VMEM_SHARED((NPAD,), jnp.float32),
    ] + [pltpu.SemaphoreType.DMA] * (3 * NBUF),
    compiler_params=_SC_PARAMS,
)

_agg_b = pl.kernel(
    _agg_b_body,
    out_type=jax.ShapeDtypeStruct((NC * NPAD, DIM), jnp.float32),
    mesh=_SC_MESH,
    scratch_types=[
        pltpu.VMEM((NCH, CHUNK), jnp.int32),
        pltpu.VMEM((NCH, CHUNK), jnp.int32),
        pltpu.VMEM((NBUF, CHUNK, DIM), jnp.float32),
        pltpu.VMEM_SHARED((N, DIM), jnp.float32),
        pltpu.VMEM_SHARED((NPAD, DIM), jnp.float32),
    ] + [pltpu.SemaphoreType.DMA] * (2 * NBUF),
    compiler_params=_SC_PARAMS,
)

_proj1 = pl.pallas_call(
    _proj1_body,
    out_shape=[
        jax.ShapeDtypeStruct((N, DIM), jnp.float32),
        jax.ShapeDtypeStruct((N, DIM), jnp.float32),
    ],
)

_combine2 = pl.pallas_call(
    _combine2_body,
    out_shape=[
        jax.ShapeDtypeStruct((N, DIM), jnp.float32),
        jax.ShapeDtypeStruct((N, DIM), jnp.float32),
        jax.ShapeDtypeStruct((N, 1), jnp.float32),
    ],
)

_final = pl.pallas_call(
    _final_body,
    out_shape=jax.ShapeDtypeStruct((G, OUT), jnp.float32),
)


@jax.jit
def kernel(x, edge_index, batch, W1_l, W1_r, W2_l, W2_r, Wfc):
    src = edge_index[0]
    dst = edge_index[1]
    pad = EP - E
    src_p = jnp.concatenate([src, jnp.zeros((pad,), jnp.int32)])
    dst_p = jnp.concatenate([dst, jnp.full((pad,), N, jnp.int32)])
    src3 = src_p.reshape(NW, NCH, CHUNK)
    dst3 = dst_p.reshape(NW, NCH, CHUNK)
    zeros_pad = jnp.zeros((NPAD, DIM), jnp.float32)
    w1 = jnp.concatenate([W1_l, W1_r], axis=0)
    w2 = jnp.concatenate([W2_l, W2_r], axis=0)

    y_l, y_r = _proj1(x, w1)
    rows1, cntp = _agg_a(y_l, src3, dst3, zeros_pad)
    rows1 = rows1.reshape(NC, NPAD, DIM)
    cntp = cntp.reshape(NC, NPAD)
    z_l, z_r, invc = _combine2(rows1, cntp, y_r, w2)
    rows2 = _agg_b(z_l, src3, dst3, zeros_pad).reshape(NC, NPAD, DIM)
    return _final(rows2, invc, z_r, batch.reshape(1, N), Wfc)


# speedup vs baseline: 27.5168x; 1.1044x over previous
import functools

import jax
import jax.numpy as jnp
from jax import lax
from jax.experimental import pallas as pl
from jax.experimental.pallas import tpu as pltpu
from jax.experimental.pallas import tpu_sc as plsc

N = 10000
E = 320000
F_IN = 128
DIM = 16
OUT = 10
G = 64

NC = 2
NS = 16
NW = NC * NS
CHUNK = 128
NCH = 80
EPW = NCH * CHUNK
EP = NW * EPW
NPAD = 10240
RPS = NPAD // NS


def _proj1_body(x_ref, w_ref, yl_ref, yr_ref):
    y = lax.dot_general(x_ref[...], w_ref[...], (((1,), (1,)), ((), ())),
                        preferred_element_type=jnp.float32)
    yl_ref[...] = y[:, :DIM]
    yr_ref[...] = y[:, DIM:]


def _final_body(acc_ref, cntp_ref, h_ref, b_ref, w2t_ref, wfc_ref, out_ref):
    cnt = cntp_ref[0] + cntp_ref[1]
    invc = (1.0 / jnp.maximum(cnt, 1.0))[:N, None]
    agg2 = (acc_ref[0] + acc_ref[1])[:N] * invc
    hh = jnp.concatenate([agg2, h_ref[:N]], axis=1)
    h2 = lax.dot_general(hh, w2t_ref[...], (((1,), (0,)), ((), ())),
                         preferred_element_type=jnp.float32)
    onehot = (lax.broadcasted_iota(jnp.int32, (G, N), 0) == b_ref[...]
              ).astype(jnp.float32)
    sums = lax.dot_general(onehot, h2, (((1,), (0,)), ((), ())),
                           preferred_element_type=jnp.float32)
    cnt_g = jnp.sum(onehot, axis=1)
    pooled = sums * (1.0 / jnp.maximum(cnt_g, 1.0))[:, None]
    logits = lax.dot_general(pooled, wfc_ref[...], (((1,), (1,)), ((), ())),
                             preferred_element_type=jnp.float32)
    out_ref[...] = jax.nn.sigmoid(logits)


_SC_MESH = plsc.VectorSubcoreMesh(core_axis_name="c", subcore_axis_name="s")
_SC_PARAMS = pltpu.CompilerParams(use_tc_tiling_on_sc=False)


NBUF = 4


def _edge_loop(y_tbl, dummy_hbm, idx_src, idx_dst, rows, acc,
               gsems, ssems, ones_buf=None, cnt_acc=None, csems=None):
    for b in range(NBUF):
        pltpu.async_copy(y_tbl.at[idx_src.at[b]], rows.at[b], gsems[b])

    gd = dummy_hbm.at[pl.ds(0, CHUNK)]

    def step(g, carry):
        for b in range(NBUF):
            j = g * NBUF + b
            pltpu.make_async_copy(gd, rows.at[b], gsems[b]).wait()
            pltpu.async_copy(rows.at[b], acc.at[idx_dst.at[j]], ssems[b],
                             add=True)
            if cnt_acc is not None:
                pltpu.async_copy(ones_buf, cnt_acc.at[idx_dst.at[j]],
                                 csems[b], add=True)
        for b in range(NBUF):
            j = g * NBUF + b
            pltpu.make_async_copy(rows.at[b], acc.at[idx_dst.at[j]],
                                  ssems[b]).wait()
            if cnt_acc is not None:
                pltpu.make_async_copy(ones_buf, cnt_acc.at[idx_dst.at[j]],
                                      csems[b]).wait()
            jn = jnp.minimum(j + NBUF, NCH - 1)
            pltpu.async_copy(y_tbl.at[idx_src.at[jn]], rows.at[b], gsems[b])
        return carry

    lax.fori_loop(0, NCH // NBUF, step, 0)
    for b in range(NBUF):
        pltpu.make_async_copy(gd, rows.at[b], gsems[b]).wait()


def _agg_a_body(y_hbm, src3, dst3, zeros_hbm, rows_out, cnt_out,
                idx_src, idx_dst, ones_buf, zbuf, rows, y_sh, acc, cnt_acc,
                *sems):
    c = lax.axis_index("c")
    s = lax.axis_index("s")
    wid = c * NS + s
    pltpu.sync_copy(src3.at[wid], idx_src)
    pltpu.sync_copy(dst3.at[wid], idx_dst)
    pltpu.sync_copy(y_hbm.at[pl.ds(s * (N // NS), N // NS)],
                    y_sh.at[pl.ds(s * (N // NS), N // NS)])
    pltpu.sync_copy(zeros_hbm.at[pl.ds(s * RPS, RPS)],
                    acc.at[pl.ds(s * RPS, RPS)])
    z16 = jnp.zeros((16,), jnp.float32)
    ones16 = jnp.ones((16,), jnp.float32)

    def fill(i, carry):
        zbuf[pl.ds(i * 16, 16)] = z16
        return carry
    lax.fori_loop(0, RPS // 16, fill, 0)

    def fill1(i, carry):
        ones_buf[pl.ds(i * 16, 16)] = ones16
        return carry
    lax.fori_loop(0, CHUNK // 16, fill1, 0)
    pltpu.sync_copy(zbuf, cnt_acc.at[pl.ds(s * RPS, RPS)])
    plsc.subcore_barrier()

    _edge_loop(y_sh, y_hbm, idx_src, idx_dst, rows, acc,
               sems[:NBUF], sems[NBUF:2 * NBUF],
               ones_buf=ones_buf, cnt_acc=cnt_acc, csems=sems[2 * NBUF:])
    plsc.subcore_barrier()

    r0 = s * RPS
    pltpu.sync_copy(acc.at[pl.ds(r0, RPS)],
                    rows_out.at[pl.ds(c * NPAD + r0, RPS)])
    pltpu.sync_copy(cnt_acc.at[pl.ds(r0, RPS)],
                    cnt_out.at[pl.ds(c * NPAD + r0, RPS)])


def _agg_b_body(acc1p, cntp, yr_hbm, src3, dst3, zeros_hbm, rows_out, h_out,
                idx_src, idx_dst, rows, a0, a1, yr2d, c0, c1, y_sh, acc,
                *sems):
    c = lax.axis_index("c")
    s = lax.axis_index("s")
    wid = c * NS + s
    pltpu.sync_copy(src3.at[wid], idx_src)
    pltpu.sync_copy(dst3.at[wid], idx_dst)
    r0 = s * RPS
    pltpu.sync_copy(acc1p.at[pl.ds(r0, RPS)], a0)
    pltpu.sync_copy(acc1p.at[pl.ds(NPAD + r0, RPS)], a1)
    pltpu.sync_copy(cntp.at[pl.ds(r0, RPS)], c0)
    pltpu.sync_copy(cntp.at[pl.ds(NPAD + r0, RPS)], c1)
    pltpu.sync_copy(yr_hbm.at[pl.ds(r0, RPS)], yr2d)
    pltpu.sync_copy(zeros_hbm.at[pl.ds(r0, RPS)], acc.at[pl.ds(r0, RPS)])

    def hgrp(g, carry):
        cv = c0[pl.ds(g * 16, 16)] + c1[pl.ds(g * 16, 16)]
        inv16 = 1.0 / jnp.maximum(cv, 1.0)
        for k in range(16):
            r = g * 16 + k
            hv = jnp.maximum((a0[r] + a1[r]) * inv16[k] + yr2d[r], 0.0)
            a0[r] = hv
        return carry
    lax.fori_loop(0, RPS // 16, hgrp, 0)
    pltpu.sync_copy(a0, y_sh.at[pl.ds(r0, RPS)])

    @pl.when(c == 0)
    def _():
        pltpu.sync_copy(a0, h_out.at[pl.ds(r0, RPS)])
    plsc.subcore_barrier()

    _edge_loop(y_sh, yr_hbm, idx_src, idx_dst, rows, acc,
               sems[:NBUF], sems[NBUF:])
    plsc.subcore_barrier()

    pltpu.sync_copy(acc.at[pl.ds(r0, RPS)],
                    rows_out.at[pl.ds(c * NPAD + r0, RPS)])


_agg_a = pl.kernel(
    _agg_a_body,
    out_type=[
        jax.ShapeDtypeStruct((NC * NPAD, DIM), jnp.float32),
        jax.ShapeDtypeStruct((NC * NPAD,), jnp.float32),
    ],
    mesh=_SC_MESH,
    scratch_types=[
        pltpu.VMEM((NCH, CHUNK), jnp.int32),
        pltpu.VMEM((NCH, CHUNK), jnp.int32),
        pltpu.VMEM((CHUNK,), jnp.float32),
        pltpu.VMEM((RPS,), jnp.float32),
        pltpu.VMEM((NBUF, CHUNK, DIM), jnp.float32),
        pltpu.VMEM_SHARED((N, DIM), jnp.float32),
        pltpu.VMEM_SHARED((NPAD, DIM), jnp.float32),
        pltpu.VMEM_SHARED((NPAD,), jnp.float32),
    ] + [pltpu.SemaphoreType.DMA] * (3 * NBUF),
    compiler_params=_SC_PARAMS,
)

_agg_b = pl.kernel(
    _agg_b_body,
    out_type=[
        jax.ShapeDtypeStruct((NC * NPAD, DIM), jnp.float32),
        jax.ShapeDtypeStruct((NPAD, DIM), jnp.float32),
    ],
    mesh=_SC_MESH,
    scratch_types=[
        pltpu.VMEM((NCH, CHUNK), jnp.int32),
        pltpu.VMEM((NCH, CHUNK), jnp.int32),
        pltpu.VMEM((NBUF, CHUNK, DIM), jnp.float32),
        pltpu.VMEM((RPS, DIM), jnp.float32),
        pltpu.VMEM((RPS, DIM), jnp.float32),
        pltpu.VMEM((RPS, DIM), jnp.float32),
        pltpu.VMEM((RPS,), jnp.float32),
        pltpu.VMEM((RPS,), jnp.float32),
        pltpu.VMEM_SHARED((NPAD, DIM), jnp.float32),
        pltpu.VMEM_SHARED((NPAD, DIM), jnp.float32),
    ] + [pltpu.SemaphoreType.DMA] * (2 * NBUF),
    compiler_params=_SC_PARAMS,
)

_proj1 = pl.pallas_call(
    _proj1_body,
    out_shape=[
        jax.ShapeDtypeStruct((N, DIM), jnp.float32),
        jax.ShapeDtypeStruct((N, DIM), jnp.float32),
    ],
)

_final = pl.pallas_call(
    _final_body,
    out_shape=jax.ShapeDtypeStruct((G, OUT), jnp.float32),
)


@jax.jit
def kernel(x, edge_index, batch, W1_l, W1_r, W2_l, W2_r, Wfc):
    src = edge_index[0]
    dst = edge_index[1]
    pad = EP - E
    src_p = jnp.concatenate([src, jnp.zeros((pad,), jnp.int32)])
    dst_p = jnp.concatenate([dst, jnp.full((pad,), N, jnp.int32)])
    src3 = src_p.reshape(NW, NCH, CHUNK)
    dst3 = dst_p.reshape(NW, NCH, CHUNK)
    zeros_pad = jnp.zeros((NPAD, DIM), jnp.float32)
    w1 = jnp.concatenate([W1_l, W1_r], axis=0)
    w2t = jnp.concatenate([W2_l.T, W2_r.T], axis=0)

    y_l, y_r = _proj1(x, w1)
    yr_pad = jnp.pad(y_r, ((0, NPAD - N), (0, 0)))
    rows1, cntp = _agg_a(y_l, src3, dst3, zeros_pad)
    rows2, h = _agg_b(rows1, cntp, yr_pad, src3, dst3, zeros_pad)
    return _final(rows2.reshape(NC, NPAD, DIM), cntp.reshape(NC, NPAD), h,
                  batch.reshape(1, N), w2t, Wfc)
